# baseline (speedup 1.0000x reference)
import functools

import jax
import jax.numpy as jnp
from jax import lax
from jax.experimental import pallas as pl
from jax.experimental.pallas import tpu as pltpu
from jax.experimental.pallas import tpu_sc as plsc

_NC, _NS = 2, 16
_NW = _NC * _NS


def _sc_gather(table, ids):
    V, E = table.shape
    N = ids.shape[0]
    per_w = N // _NW
    CH = 512
    n_ch = per_w // CH
    mesh = plsc.VectorSubcoreMesh(core_axis_name="c", subcore_axis_name="s")

    @functools.partial(
        pl.kernel,
        out_type=jax.ShapeDtypeStruct((N, E), jnp.float32),
        mesh=mesh,
        scratch_types=[
            pltpu.VMEM((per_w,), jnp.int32),
            pltpu.VMEM((CH, E), jnp.float32),
            pltpu.SemaphoreType.DMA,
        ],
    )
    def k(table_hbm, idx_hbm, out_hbm, idx_v, rows_v, sem):
        wid = lax.axis_index("s") * _NC + lax.axis_index("c")
        base = wid * per_w
        pltpu.sync_copy(idx_hbm.at[pl.ds(base, per_w)], idx_v)

        @pl.loop(0, n_ch)
        def _(i):
            start = i * CH
            pltpu.async_copy(
                table_hbm.at[idx_v.at[pl.ds(start, CH)]], rows_v, sem
            ).wait()
            pltpu.sync_copy(rows_v, out_hbm.at[pl.ds(base + start, CH)])

    return k(table, ids)


def _tc_body(we_ref, ttf_ref, w_ref, csum_ref, dg_ref, out_ref):
    BB, S, E = we_ref.shape
    H = out_ref.shape[2]
    we2 = we_ref[...].reshape(BB * S, E)
    zrow = jnp.zeros((1, E), jnp.float32)
    left = jnp.concatenate([we2[1:], zrow], axis=0)
    right = jnp.concatenate([zrow, we2[:-1]], axis=0)
    r = lax.broadcasted_iota(jnp.int32, (BB * S, 1), 0) % S
    left = jnp.where(r == (S - 1), 0.0, left)
    right = jnp.where(r == 0, 0.0, right)
    x = jnp.dot(we2, w_ref[E:2 * E], preferred_element_type=jnp.float32)
    x = x + jnp.dot(left, w_ref[:E], preferred_element_type=jnp.float32)
    x = x + jnp.dot(right, w_ref[2 * E:], preferred_element_type=jnp.float32)
    x3 = x.reshape(BB, S, H)
    out_ref[...] = x3 + csum_ref[...][None] + ttf_ref[...] * dg_ref[...]


def _tc_chunk_body(buf_ref, we_ref, ttf_ref, w_ref, csum_ref, dg_ref, out_ref):
    del buf_ref
    _tc_body(we_ref, ttf_ref, w_ref, csum_ref, dg_ref, out_ref)


def _tc_embed_chunk(buf, we3, ttf, Wg, csum, dg, c, BB):
    B, S, H = buf.shape
    BC, _, E = we3.shape
    nb = BC // BB
    off = c * nb
    return pl.pallas_call(
        _tc_chunk_body,
        grid=(nb,),
        in_specs=[
            pl.BlockSpec(memory_space=pl.ANY),
            pl.BlockSpec((BB, S, E), lambda j: (j, 0, 0)),
            pl.BlockSpec((BB, S, 1), lambda j: (off + j, 0, 0)),
            pl.BlockSpec((3 * E, H), lambda j: (0, 0)),
            pl.BlockSpec((S, H), lambda j: (0, 0)),
            pl.BlockSpec((1, H), lambda j: (0, 0)),
        ],
        out_specs=pl.BlockSpec((BB, S, H), lambda j: (off + j, 0, 0)),
        out_shape=jax.ShapeDtypeStruct((B, S, H), jnp.float32),
        input_output_aliases={0: 0},
    )(buf, we3, ttf, Wg, csum, dg)


def _tc_embed_first(we3, ttf, Wg, csum, dg, B, BB):
    BC, S, E = we3.shape
    H = csum.shape[1]
    nb = BC // BB
    return pl.pallas_call(
        _tc_body,
        grid=(nb,),
        in_specs=[
            pl.BlockSpec((BB, S, E), lambda j: (j, 0, 0)),
            pl.BlockSpec((BB, S, 1), lambda j: (j, 0, 0)),
            pl.BlockSpec((3 * E, H), lambda j: (0, 0)),
            pl.BlockSpec((S, H), lambda j: (0, 0)),
            pl.BlockSpec((1, H), lambda j: (0, 0)),
        ],
        out_specs=pl.BlockSpec((BB, S, H), lambda j: (j, 0, 0)),
        out_shape=jax.ShapeDtypeStruct((B, S, H), jnp.float32),
    )(we3, ttf, Wg, csum, dg)


def kernel(input_ids, token_type_ids, position_ids, word_table, pos_table,
           type_table, W, b, gamma, beta):
    B, S = input_ids.shape
    V, E = word_table.shape
    H = pos_table.shape[1]

    pe = jnp.take(pos_table, position_ids[0], axis=0)
    Wg = W.T * gamma[None, :]
    csum = (pe + b[None, :] + type_table[0][None, :]) * gamma[None, :] \
        + beta[None, :]
    dg = ((type_table[1] - type_table[0]) * gamma).reshape(1, H)
    ttf = token_type_ids.astype(jnp.float32).reshape(B, S, 1)

    NCH = 4
    BB = 8
    BC = B // NCH
    ids = input_ids.reshape(NCH, BC * S).astype(jnp.int32)
    wes = [_sc_gather(word_table, ids[c]).reshape(BC, S, E)
           for c in range(NCH)]
    buf = _tc_embed_first(wes[0], ttf[:BC], Wg, csum, dg, B, BB)
    for c in range(1, NCH):
        buf = _tc_embed_chunk(buf, wes[c], ttf, Wg, csum, dg, c, BB)
    return buf


# speedup vs baseline: 1.0444x; 1.0444x over previous
import functools

import jax
import jax.numpy as jnp
from jax import lax
from jax.experimental import pallas as pl
from jax.experimental.pallas import tpu as pltpu
from jax.experimental.pallas import tpu_sc as plsc

_NC, _NS = 2, 16
_NW = _NC * _NS


def _sc_gather(table, ids, BC, S):
    V, E = table.shape
    spw = BC // _NW
    mesh = plsc.VectorSubcoreMesh(core_axis_name="c", subcore_axis_name="s")

    @functools.partial(
        pl.kernel,
        out_type=jax.ShapeDtypeStruct((BC, S, E), jnp.float32),
        mesh=mesh,
        scratch_types=[
            pltpu.VMEM((spw * S,), jnp.int32),
            pltpu.VMEM((S, E), jnp.float32),
            pltpu.SemaphoreType.DMA,
        ],
    )
    def k(table_hbm, idx_hbm, out_hbm, idx_v, rows_v, sem):
        wid = lax.axis_index("s") * _NC + lax.axis_index("c")
        base = wid * spw
        pltpu.sync_copy(idx_hbm.at[pl.ds(base * S, spw * S)], idx_v)

        @pl.loop(0, spw)
        def _(i):
            pltpu.async_copy(
                table_hbm.at[idx_v.at[pl.ds(i * S, S)]], rows_v, sem
            ).wait()
            pltpu.sync_copy(rows_v, out_hbm.at[base + i])

    return k(table, ids)


def _tc_body(we_ref, ttf_ref, w_ref, csum_ref, dg_ref, out_ref):
    BB, S, E = we_ref.shape
    H = out_ref.shape[2]
    we2 = we_ref[...].reshape(BB * S, E)
    zrow = jnp.zeros((1, E), jnp.float32)
    left = jnp.concatenate([we2[1:], zrow], axis=0)
    right = jnp.concatenate([zrow, we2[:-1]], axis=0)
    r = lax.broadcasted_iota(jnp.int32, (BB * S, 1), 0) % S
    left = jnp.where(r == (S - 1), 0.0, left)
    right = jnp.where(r == 0, 0.0, right)
    x = jnp.dot(we2, w_ref[E:2 * E], preferred_element_type=jnp.float32)
    x = x + jnp.dot(left, w_ref[:E], preferred_element_type=jnp.float32)
    x = x + jnp.dot(right, w_ref[2 * E:], preferred_element_type=jnp.float32)
    x3 = x.reshape(BB, S, H)
    out_ref[...] = x3 + csum_ref[...][None] + ttf_ref[...] * dg_ref[...]


def _tc_chunk_body(buf_ref, we_ref, ttf_ref, w_ref, csum_ref, dg_ref, out_ref):
    del buf_ref
    _tc_body(we_ref, ttf_ref, w_ref, csum_ref, dg_ref, out_ref)


def _tc_embed_chunk(buf, we3, ttf, Wg, csum, dg, c, BB, B):
    BC, S, E = we3.shape
    H = csum.shape[1]
    nb = BC // BB
    off = c * nb
    specs = [
        pl.BlockSpec((BB, S, E), lambda j: (j, 0, 0)),
        pl.BlockSpec((BB, S, 1), lambda j: (off + j, 0, 0)),
        pl.BlockSpec((3 * E, H), lambda j: (0, 0)),
        pl.BlockSpec((S, H), lambda j: (0, 0)),
        pl.BlockSpec((1, H), lambda j: (0, 0)),
    ]
    if buf is None:
        return pl.pallas_call(
            _tc_body,
            grid=(nb,),
            in_specs=specs,
            out_specs=pl.BlockSpec((BB, S, H), lambda j: (off + j, 0, 0)),
            out_shape=jax.ShapeDtypeStruct((B, S, H), jnp.float32),
        )(we3, ttf, Wg, csum, dg)
    return pl.pallas_call(
        _tc_chunk_body,
        grid=(nb,),
        in_specs=[pl.BlockSpec(memory_space=pl.ANY)] + specs,
        out_specs=pl.BlockSpec((BB, S, H), lambda j: (off + j, 0, 0)),
        out_shape=jax.ShapeDtypeStruct((B, S, H), jnp.float32),
        input_output_aliases={0: 0},
    )(buf, we3, ttf, Wg, csum, dg)


def kernel(input_ids, token_type_ids, position_ids, word_table, pos_table,
           type_table, W, b, gamma, beta):
    B, S = input_ids.shape
    V, E = word_table.shape
    H = pos_table.shape[1]

    pe = jnp.take(pos_table, position_ids[0], axis=0)
    Wg = W.T * gamma[None, :]
    csum = (pe + b[None, :] + type_table[0][None, :]) * gamma[None, :] \
        + beta[None, :]
    dg = ((type_table[1] - type_table[0]) * gamma).reshape(1, H)
    ttf = token_type_ids.astype(jnp.float32).reshape(B, S, 1)
    NCH = 4
    BB = 8
    BC = B // NCH
    ids = input_ids.astype(jnp.int32).reshape(NCH, BC * S)
    wes = [_sc_gather(word_table, ids[c], BC, S) for c in range(NCH)]
    buf = None
    for c in range(NCH):
        buf = _tc_embed_chunk(buf, wes[c], ttf, Wg, csum, dg, c, BB, B)
    return buf


# speedup vs baseline: 1.2454x; 1.1925x over previous
import functools

import jax
import jax.numpy as jnp
from jax import lax
from jax.experimental import pallas as pl
from jax.experimental.pallas import tpu as pltpu
from jax.experimental.pallas import tpu_sc as plsc

_NC, _NS = 2, 16
_NW = _NC * _NS


def _sc_gather(table, ids, BC, S):
    V, E = table.shape
    spw = BC // _NW
    mesh = plsc.VectorSubcoreMesh(core_axis_name="c", subcore_axis_name="s")

    @functools.partial(
        pl.kernel,
        out_type=jax.ShapeDtypeStruct((BC, S, E), jnp.float32),
        mesh=mesh,
        scratch_types=[
            pltpu.VMEM((spw * S,), jnp.int32),
            pltpu.VMEM((S, E), jnp.float32),
            pltpu.SemaphoreType.DMA,
        ],
    )
    def k(table_hbm, idx_hbm, out_hbm, idx_v, rows_v, sem):
        wid = lax.axis_index("s") * _NC + lax.axis_index("c")
        base = wid * spw
        pltpu.sync_copy(idx_hbm.at[pl.ds(base * S, spw * S)], idx_v)

        @pl.loop(0, spw)
        def _(i):
            pltpu.async_copy(
                table_hbm.at[idx_v.at[pl.ds(i * S, S)]], rows_v, sem
            ).wait()
            pltpu.sync_copy(rows_v, out_hbm.at[base + i])

    return k(table, ids)


def _tc_body(we_ref, tt_ref, w_ref, csum_ref, dg_ref, out_ref):
    BB, S, E = we_ref.shape
    H = out_ref.shape[2]
    we2 = we_ref[...].reshape(BB * S, E)
    zrow = jnp.zeros((1, E), jnp.float32)
    left = jnp.concatenate([we2[1:], zrow], axis=0)
    right = jnp.concatenate([zrow, we2[:-1]], axis=0)
    r = lax.broadcasted_iota(jnp.int32, (BB * S, 1), 0) % S
    left = jnp.where(r == (S - 1), 0.0, left)
    right = jnp.where(r == 0, 0.0, right)
    x = jnp.dot(we2, w_ref[E:2 * E], preferred_element_type=jnp.float32)
    x = x + jnp.dot(left, w_ref[:E], preferred_element_type=jnp.float32)
    x = x + jnp.dot(right, w_ref[2 * E:], preferred_element_type=jnp.float32)
    x3 = x.reshape(BB, S, H)
    acc = x3 + csum_ref[...][None]
    tt_t = tt_ref[...].astype(jnp.float32).T
    dg = dg_ref[...]
    for bb in range(BB):
        out_ref[bb] = acc[bb] + tt_t[:, bb:bb + 1] * dg


def _tc_chunk_body(buf_ref, we_ref, tt_ref, w_ref, csum_ref, dg_ref, out_ref):
    del buf_ref
    _tc_body(we_ref, tt_ref, w_ref, csum_ref, dg_ref, out_ref)


def _tc_embed_chunk(buf, we3, tt, Wg, csum, dg, off, BB, B):
    BC, S, E = we3.shape
    H = csum.shape[1]
    nb = BC // BB
    specs = [
        pl.BlockSpec((BB, S, E), lambda j: (j, 0, 0)),
        pl.BlockSpec((BB, S), lambda j: (off + j, 0)),
        pl.BlockSpec((3 * E, H), lambda j: (0, 0)),
        pl.BlockSpec((S, H), lambda j: (0, 0)),
        pl.BlockSpec((1, H), lambda j: (0, 0)),
    ]
    if buf is None:
        return pl.pallas_call(
            _tc_body,
            grid=(nb,),
            in_specs=specs,
            out_specs=pl.BlockSpec((BB, S, H), lambda j: (off + j, 0, 0)),
            out_shape=jax.ShapeDtypeStruct((B, S, H), jnp.float32),
        )(we3, tt, Wg, csum, dg)
    return pl.pallas_call(
        _tc_chunk_body,
        grid=(nb,),
        in_specs=[pl.BlockSpec(memory_space=pl.ANY)] + specs,
        out_specs=pl.BlockSpec((BB, S, H), lambda j: (off + j, 0, 0)),
        out_shape=jax.ShapeDtypeStruct((B, S, H), jnp.float32),
        input_output_aliases={0: 0},
    )(buf, we3, tt, Wg, csum, dg)


def kernel(input_ids, token_type_ids, position_ids, word_table, pos_table,
           type_table, W, b, gamma, beta):
    B, S = input_ids.shape
    V, E = word_table.shape
    H = pos_table.shape[1]

    pe = jnp.take(pos_table, position_ids[0], axis=0)
    Wg = W.T * gamma[None, :]
    csum = (pe + b[None, :] + type_table[0][None, :]) * gamma[None, :] \
        + beta[None, :]
    dg = ((type_table[1] - type_table[0]) * gamma).reshape(1, H)
    tt = token_type_ids.astype(jnp.int32)

    chunks = [32, 64, 64, 96] if B == 256 else [B // 4] * 4
    BB = 8
    ids = input_ids.astype(jnp.int32).reshape(B * S)
    wes = []
    start = 0
    for BC in chunks:
        wes.append(_sc_gather(
            word_table, lax.slice(ids, (start * S,), ((start + BC) * S,)),
            BC, S))
        start += BC
    buf = None
    start = 0
    for BC, we3 in zip(chunks, wes):
        buf = _tc_embed_chunk(buf, we3, tt, Wg, csum, dg, start // BB, BB, B)
        start += BC
    return buf
